# baseline (speedup 1.0000x reference)
import functools

import jax
import jax.numpy as jnp
from jax import lax
from jax.experimental import pallas as pl
from jax.experimental.pallas import tpu as pltpu
from jax.experimental.pallas import tpu_sc as plsc

_NW = 32
_RC = 16
_LANES = 16
_S_SC = 256
_CHUNK = 256


def _sc_pool_body(x_hbm, out_hbm, buf, acc, sem0, sem1, *, S):
    D = acc.shape[0]
    B = x_hbm.shape[0] // S
    wpb = _NW // B
    nrows = _S_SC // wpb
    nch = nrows // _RC
    wid = lax.axis_index("s") * 2 + lax.axis_index("c")
    row0 = (wid // wpb) * S + (wid % wpb) * nrows
    sems = (sem0, sem1)

    def _copy(c, p):
        return pltpu.make_async_copy(
            x_hbm.at[pl.ds(row0 + c * _RC, _RC), :], buf.at[p], sems[p])

    def _zero(j, _):
        acc[pl.ds(j * _LANES, _LANES)] = jnp.zeros((_LANES,), jnp.float32)
        return 0

    lax.fori_loop(0, D // _LANES, _zero, 0, unroll=True)

    _copy(0, 0).start()
    _copy(1, 1).start()

    def _chunk(g, _):
        for p in range(2):
            c = g * 2 + p
            _copy(c, p).wait()

            @pl.when(c + 2 < nch)
            def _start_next():
                _copy(c + 2, p).start()

            def _accum(j, _):
                sl = pl.ds(j * _LANES, _LANES)
                vals = [buf[p, r, sl] for r in range(_RC)]
                while len(vals) > 1:
                    vals = [vals[i] + vals[i + 1]
                            for i in range(0, len(vals), 2)]
                acc[sl] = acc[sl] + vals[0]
                return 0

            lax.fori_loop(0, D // _LANES, _accum, 0, unroll=4)
        return 0

    lax.fori_loop(0, nch // 2, _chunk, 0)
    pltpu.sync_copy(acc, out_hbm.at[wid, :])


def _sc_pool(x2d, S, D):
    mesh = plsc.VectorSubcoreMesh(core_axis_name="c", subcore_axis_name="s")
    k = functools.partial(
        pl.kernel,
        mesh=mesh,
        out_type=jax.ShapeDtypeStruct((_NW, D), jnp.float32),
        scratch_types=[
            pltpu.VMEM((2, _RC, D), jnp.float32),
            pltpu.VMEM((D,), jnp.float32),
            pltpu.SemaphoreType.DMA,
            pltpu.SemaphoreType.DMA,
        ],
    )(functools.partial(_sc_pool_body, S=S))
    return k(x2d)


def _tc_pool_kernel(x_ref, acc_ref):
    c = pl.program_id(0)

    @pl.when(c == 0)
    def _init():
        acc_ref[:] = jnp.zeros_like(acc_ref)

    acc_ref[:] += jnp.sum(x_ref[:], axis=1)


def _finish_kernel(task_id_ref, mode_id_ref, ps_ref, tc_ref, w_ref, tb_ref,
                   mb_ref, idx_ref, wgt_ref, *, S):
    B = idx_ref.shape[0]
    E = w_ref.shape[0]
    T = tb_ref.shape[0]
    M = mb_ref.shape[0]
    G = ps_ref.shape[0] // B

    ps = ps_ref[:]
    pooled = (jnp.concatenate(
        [jnp.sum(ps[G * b:G * (b + 1)], axis=0, keepdims=True)
         for b in range(B)], axis=0) + tc_ref[:]) * (1.0 / S)
    logits = jax.lax.dot_general(
        pooled, w_ref[:], (((1,), (1,)), ((), ())),
        preferred_element_type=jnp.float32)

    t_iota = jax.lax.broadcasted_iota(jnp.int32, (1, T), 1)
    m_iota = jax.lax.broadcasted_iota(jnp.int32, (1, M), 1)
    oh_t = jnp.concatenate(
        [(t_iota == task_id_ref[b]).astype(jnp.float32) for b in range(B)],
        axis=0)
    oh_m = jnp.concatenate(
        [(m_iota == mode_id_ref[b]).astype(jnp.float32) for b in range(B)],
        axis=0)
    logits = logits + oh_t @ tb_ref[:] + oh_m @ mb_ref[:]

    e_iota = jax.lax.broadcasted_iota(jnp.int32, (B, E), 1)
    m1 = jnp.max(logits, axis=1, keepdims=True)
    i1 = jnp.min(jnp.where(logits == m1, e_iota, E), axis=1, keepdims=True)
    masked = jnp.where(e_iota == i1, -jnp.inf, logits)
    m2 = jnp.max(masked, axis=1, keepdims=True)
    i2 = jnp.min(jnp.where(masked == m2, e_iota, E), axis=1, keepdims=True)

    idx_ref[:] = jnp.concatenate([i1, i2], axis=1)
    r = jnp.exp(m2 - m1)
    w1 = 1.0 / (1.0 + r)
    wgt_ref[:] = jnp.concatenate([w1, 1.0 - w1], axis=1)


@jax.jit
def _impl(x, task_id, mode_id, W, task_bias, mode_bias):
    B, S, D = x.shape
    sc_partials = _sc_pool(x.reshape(B * S, D), S, D)

    off = _S_SC // _CHUNK
    nc = (S - _S_SC) // _CHUNK
    tc_acc = pl.pallas_call(
        _tc_pool_kernel,
        grid=(nc,),
        in_specs=[pl.BlockSpec((B, _CHUNK, D), lambda c: (0, c + off, 0))],
        out_specs=pl.BlockSpec((B, D), lambda c: (0, 0)),
        out_shape=jax.ShapeDtypeStruct((B, D), jnp.float32),
        compiler_params=pltpu.CompilerParams(
            dimension_semantics=("arbitrary",)),
    )(x)

    idx, wgt = pl.pallas_call(
        functools.partial(_finish_kernel, S=S),
        in_specs=[
            pl.BlockSpec(memory_space=pltpu.SMEM),
            pl.BlockSpec(memory_space=pltpu.SMEM),
            pl.BlockSpec(sc_partials.shape, lambda: (0, 0)),
            pl.BlockSpec((B, D), lambda: (0, 0)),
            pl.BlockSpec(W.shape, lambda: (0, 0)),
            pl.BlockSpec(task_bias.shape, lambda: (0, 0)),
            pl.BlockSpec(mode_bias.shape, lambda: (0, 0)),
        ],
        out_specs=[
            pl.BlockSpec((B, 2), lambda: (0, 0)),
            pl.BlockSpec((B, 2), lambda: (0, 0)),
        ],
        out_shape=[
            jax.ShapeDtypeStruct((B, 2), jnp.int32),
            jax.ShapeDtypeStruct((B, 2), jnp.float32),
        ],
    )(task_id.astype(jnp.int32), mode_id.astype(jnp.int32),
      sc_partials, tc_acc, W, task_bias, mode_bias)
    return idx, wgt


def kernel(x, task_id, mode_id, W, task_bias, mode_bias):
    return _impl(x, task_id, mode_id, W, task_bias, mode_bias)


# speedup vs baseline: 1.2530x; 1.2530x over previous
import functools

import jax
import jax.numpy as jnp
from jax import lax
from jax.experimental import pallas as pl
from jax.experimental.pallas import tpu as pltpu

_ROWS = 128
_NBUF = 4


def _router_kernel(task_id_ref, mode_id_ref, x_hbm, w_ref, tb_ref, mb_ref,
                   idx_ref, wgt_ref, bufs, acc, s0, s1, s2, s3, *, B, S):
    D = acc.shape[1]
    nch = (B * S) // _ROWS
    cpb = S // _ROWS
    sems = (s0, s1, s2, s3)

    def _copy(c, p):
        return pltpu.make_async_copy(
            x_hbm.at[pl.ds(c * _ROWS, _ROWS), :], bufs.at[p], sems[p])

    for p in range(_NBUF):
        _copy(p, p).start()
    acc[:] = jnp.zeros_like(acc)

    def _chunk(g, _):
        for p in range(_NBUF):
            c = g * _NBUF + p
            _copy(c, p).wait()

            @pl.when(c + _NBUF < nch)
            def _start_next():
                _copy(c + _NBUF, p).start()

            v = jnp.sum(bufs[p], axis=0, keepdims=True)
            bb = c // cpb
            for b in range(B):
                @pl.when(bb == b)
                def _acc_b():
                    acc[pl.ds(b, 1), :] += v
        return 0

    lax.fori_loop(0, nch // _NBUF, _chunk, 0)

    E = w_ref.shape[0]
    T = tb_ref.shape[0]
    M = mb_ref.shape[0]

    pooled = acc[:] * (1.0 / S)
    logits = jax.lax.dot_general(
        pooled, w_ref[:], (((1,), (1,)), ((), ())),
        preferred_element_type=jnp.float32)

    t_iota = jax.lax.broadcasted_iota(jnp.int32, (1, T), 1)
    m_iota = jax.lax.broadcasted_iota(jnp.int32, (1, M), 1)
    oh_t = jnp.concatenate(
        [(t_iota == task_id_ref[b]).astype(jnp.float32) for b in range(B)],
        axis=0)
    oh_m = jnp.concatenate(
        [(m_iota == mode_id_ref[b]).astype(jnp.float32) for b in range(B)],
        axis=0)
    logits = logits + oh_t @ tb_ref[:] + oh_m @ mb_ref[:]

    e_iota = jax.lax.broadcasted_iota(jnp.int32, (B, E), 1)
    m1 = jnp.max(logits, axis=1, keepdims=True)
    i1 = jnp.min(jnp.where(logits == m1, e_iota, E), axis=1, keepdims=True)
    masked = jnp.where(e_iota == i1, -jnp.inf, logits)
    m2 = jnp.max(masked, axis=1, keepdims=True)
    i2 = jnp.min(jnp.where(masked == m2, e_iota, E), axis=1, keepdims=True)

    idx_ref[:] = jnp.concatenate([i1, i2], axis=1)
    r = jnp.exp(m2 - m1)
    w1 = 1.0 / (1.0 + r)
    wgt_ref[:] = jnp.concatenate([w1, 1.0 - w1], axis=1)


@jax.jit
def _impl(x, task_id, mode_id, W, task_bias, mode_bias):
    B, S, D = x.shape

    idx, wgt = pl.pallas_call(
        functools.partial(_router_kernel, B=B, S=S),
        in_specs=[
            pl.BlockSpec(memory_space=pltpu.SMEM),
            pl.BlockSpec(memory_space=pltpu.SMEM),
            pl.BlockSpec(memory_space=pl.ANY),
            pl.BlockSpec(W.shape, lambda: (0, 0)),
            pl.BlockSpec(task_bias.shape, lambda: (0, 0)),
            pl.BlockSpec(mode_bias.shape, lambda: (0, 0)),
        ],
        out_specs=[
            pl.BlockSpec((B, 2), lambda: (0, 0)),
            pl.BlockSpec((B, 2), lambda: (0, 0)),
        ],
        out_shape=[
            jax.ShapeDtypeStruct((B, 2), jnp.int32),
            jax.ShapeDtypeStruct((B, 2), jnp.float32),
        ],
        scratch_shapes=[
            pltpu.VMEM((_NBUF, _ROWS, D), jnp.float32),
            pltpu.VMEM((B, D), jnp.float32),
            pltpu.SemaphoreType.DMA,
            pltpu.SemaphoreType.DMA,
            pltpu.SemaphoreType.DMA,
            pltpu.SemaphoreType.DMA,
        ],
    )(task_id.astype(jnp.int32), mode_id.astype(jnp.int32),
      x.reshape(B * S, D), W, task_bias, mode_bias)
    return idx, wgt


def kernel(x, task_id, mode_id, W, task_bias, mode_bias):
    return _impl(x, task_id, mode_id, W, task_bias, mode_bias)
